# baseline (speedup 1.0000x reference)
import functools

import jax
import jax.numpy as jnp
from jax import lax
from jax.experimental import pallas as pl
from jax.experimental.pallas import tpu as pltpu
from jax.experimental.pallas import tpu_sc as plsc

N = 10000
E = 320000
D = 128
NC = 2
NS = 16
L = 16
NW = NC * NS
E_W = E // NW
KB = 80
NBLK = E_W // KB
AGG_CH = 1000


def _prologue_body(x_ref, w_ref, a_ref, c_ref, p_ref,
                   h_ref, sl_ref, sr_ref, b0_ref, g_ref):
    x = x_ref[...]
    h = jnp.maximum(jnp.dot(x, w_ref[...], preferred_element_type=jnp.float32), 0.0)
    h_ref[...] = h
    s = jnp.dot(h, a_ref[...], preferred_element_type=jnp.float32)
    sl = s[:, 0:1] + p_ref[0]
    sr = s[:, 1:2] + p_ref[1]
    sl_ref[...] = sl
    sr_ref[...] = sr
    b = jnp.dot(x, c_ref[...], preferred_element_type=jnp.float32)
    b0_ref[...] = jax.nn.sigmoid(b[:, 0:1] + p_ref[2])
    m = jnp.max(sl) + jnp.max(sr)
    g = jnp.maximum(m, 0.2 * m)
    g_ref[...] = jnp.full((1, 128), g, jnp.float32)


def _prologue(x, w_lin, a_mat, c_mat, p):
    return pl.pallas_call(
        _prologue_body,
        out_shape=(
            jax.ShapeDtypeStruct((N, D), jnp.float32),
            jax.ShapeDtypeStruct((N, 1), jnp.float32),
            jax.ShapeDtypeStruct((N, 1), jnp.float32),
            jax.ShapeDtypeStruct((N, 1), jnp.float32),
            jax.ShapeDtypeStruct((1, 128), jnp.float32),
        ),
        in_specs=[
            pl.BlockSpec(memory_space=pltpu.VMEM),
            pl.BlockSpec(memory_space=pltpu.VMEM),
            pl.BlockSpec(memory_space=pltpu.VMEM),
            pl.BlockSpec(memory_space=pltpu.VMEM),
            pl.BlockSpec(memory_space=pltpu.SMEM),
        ],
    )(x, w_lin, a_mat, c_mat, p)


def _edge_body(h_hbm, sl_hbm, sr_hbm, ii_hbm, jj_hbm, g_hbm,
               aggp_hbm, denp_hbm,
               sh_agg, sh_den, sl_v, sr_v,
               ii0, jj0, e0, rows0, ii1, jj1, e1, rows1, g_v,
               isem0, gsem0, rsem0, dsem0, isem1, gsem1, rsem1, dsem1):
    cid = lax.axis_index("c")
    sid = lax.axis_index("s")
    wid = sid * NC + cid
    bufs = ((ii0, jj0, e0, rows0, isem0, gsem0, rsem0, dsem0),
            (ii1, jj1, e1, rows1, isem1, gsem1, rsem1, dsem1))

    pltpu.sync_copy(sl_hbm, sl_v)
    pltpu.sync_copy(sr_hbm, sr_v)
    pltpu.sync_copy(g_hbm.at[pl.ds(0, L)], g_v)

    def _issue_idx(b, s):
        ii_v, jj_v = bufs[s][0], bufs[s][1]
        isem = bufs[s][4]
        off = wid * E_W + b * KB
        pltpu.async_copy(ii_hbm.at[pl.ds(off, KB)], ii_v, isem)
        pltpu.async_copy(jj_hbm.at[pl.ds(off, KB)], jj_v, isem)

    def _zrow(r, _):
        for c in range(D // L):
            rows0[r, pl.ds(c * L, L)] = jnp.zeros((L,), jnp.float32)
        return 0
    lax.fori_loop(0, KB, _zrow, 0)
    for q in range(KB // L):
        e0[pl.ds(q * L, L)] = jnp.zeros((L,), jnp.float32)

    _issue_idx(0, 0)

    nch = N // KB
    for k in range((nch + NS - 1) // NS):
        c = sid + k * NS

        @pl.when(c < nch)
        def _():
            pltpu.sync_copy(rows0, sh_agg.at[pl.ds(c * KB, KB)])
            pltpu.sync_copy(e0, sh_den.at[pl.ds(c * KB, KB)])

    plsc.subcore_barrier()

    g16 = g_v[...]

    def _process(b, s, first):
        ii_v, jj_v, e_v, rows_v, isem, gsem, rsem, dsem = bufs[s]
        o = 1 - s
        ii_o, jj_o, e_o, rows_o, isem_o, gsem_o, rsem_o, dsem_o = bufs[o]
        off = wid * E_W + b * KB
        pltpu.make_async_copy(ii_hbm.at[pl.ds(off, KB)], ii_v, isem).wait()
        pltpu.make_async_copy(jj_hbm.at[pl.ds(off, KB)], jj_v, isem).wait()
        gcp = pltpu.async_copy(h_hbm.at[jj_v], rows_v, gsem)

        for q in range(KB // L):
            ii16 = ii_v[pl.ds(q * L, L)]
            jj16 = jj_v[pl.ds(q * L, L)]
            sv = (plsc.load_gather(sl_v, [ii16]) +
                  plsc.load_gather(sr_v, [jj16]))
            a = jnp.maximum(sv, 0.2 * sv)
            e_v[pl.ds(q * L, L)] = jnp.exp(a - g16)

        gcp.wait()

        @plsc.parallel_loop(0, KB // L, unroll=5)
        def _rows16(q):
            e16 = e_v[pl.ds(q * L, L)]
            for r in range(L):
                eb = jnp.take_along_axis(
                    e16, jnp.full((L,), r, jnp.int32), axis=0,
                    mode="promise_in_bounds")
                row = q * L + r
                for c in range(D // L):
                    rows_v[row, pl.ds(c * L, L)] = (
                        rows_v[row, pl.ds(c * L, L)] * eb)

        pltpu.async_copy(rows_v, sh_agg.at[ii_v], rsem, add=True)
        pltpu.async_copy(e_v, sh_den.at[ii_v], dsem, add=True)

        if not first:
            pltpu.make_async_copy(rows_o, sh_agg.at[ii_o], rsem_o).wait()
            pltpu.make_async_copy(e_o, sh_den.at[ii_o], dsem_o).wait()

        @pl.when(b + 1 < NBLK)
        def _():
            _issue_idx(b + 1, o)

    _process(0, 0, True)

    def _pair(p, _):
        _process(1 + 2 * p, 1, False)
        _process(2 + 2 * p, 0, False)
        return 0
    lax.fori_loop(0, (NBLK - 1) // 2, _pair, 0)

    pltpu.make_async_copy(rows0, sh_agg.at[ii0], rsem0).wait()
    pltpu.make_async_copy(e0, sh_den.at[ii0], dsem0).wait()

    plsc.subcore_barrier()

    @pl.when(sid < N // AGG_CH)
    def _():
        pltpu.sync_copy(sh_agg.at[pl.ds(sid * AGG_CH, AGG_CH)],
                        aggp_hbm.at[cid, pl.ds(sid * AGG_CH, AGG_CH)])

    @pl.when(sid == 0)
    def _():
        pltpu.sync_copy(sh_den, denp_hbm.at[cid])


def _edge_kernel(h, sl, sr, ii, jj, g):
    mesh = plsc.VectorSubcoreMesh(core_axis_name="c", subcore_axis_name="s",
                                  num_cores=NC, num_subcores=NS)
    return pl.kernel(
        _edge_body,
        out_type=(
            jax.ShapeDtypeStruct((NC, N, D), jnp.float32),
            jax.ShapeDtypeStruct((NC, N), jnp.float32),
        ),
        mesh=mesh,
        compiler_params=pltpu.CompilerParams(needs_layout_passes=False),
        scratch_types=(
            [
                pltpu.VMEM_SHARED((N, D), jnp.float32),
                pltpu.VMEM_SHARED((N,), jnp.float32),
                pltpu.VMEM((N,), jnp.float32),
                pltpu.VMEM((N,), jnp.float32),
            ]
            + 2 * [
                pltpu.VMEM((KB,), jnp.int32),
                pltpu.VMEM((KB,), jnp.int32),
                pltpu.VMEM((KB,), jnp.float32),
                pltpu.VMEM((KB, D), jnp.float32),
            ]
            + [pltpu.VMEM((L,), jnp.float32)]
            + 8 * [pltpu.SemaphoreType.DMA]
        ),
    )(h, sl, sr, ii, jj, g)


def _combine_body(aggp_ref, denp_ref, h_ref, b0_ref, out_ref):
    agg = aggp_ref[0] + aggp_ref[1]
    den = denp_ref[0] + denp_ref[1] + 1e-16
    b0 = b0_ref[...][:, None]
    out_ref[...] = b0 * (agg / den[:, None]) + (1.0 - b0) * h_ref[...]


def _combine(aggp, denp, h, b0):
    return pl.pallas_call(
        _combine_body,
        out_shape=jax.ShapeDtypeStruct((N, D), jnp.float32),
    )(aggp, denp, h, b0)


def _latte(x, x_index, edge_index, W_lin, conv_w, conv_b,
           attn_l_w, attn_l_b, attn_r_w, attn_r_b):
    p = jnp.stack([
        jnp.asarray(attn_l_b, jnp.float32),
        jnp.asarray(attn_r_b, jnp.float32),
        (conv_b[0] - conv_b[1]).astype(jnp.float32),
    ])
    a_mat = jnp.zeros((D, 128), jnp.float32)
    a_mat = a_mat.at[:, 0].set(attn_l_w).at[:, 1].set(attn_r_w)
    c_mat = jnp.zeros((D, 128), jnp.float32)
    c_mat = c_mat.at[:, 0].set(conv_w[0] - conv_w[1])
    h, sl, sr, b0, g = _prologue(x, W_lin, a_mat, c_mat, p)
    ii = edge_index[0]
    jj = edge_index[1]
    aggp, denp = _edge_kernel(h, sl.reshape(N), sr.reshape(N), ii, jj,
                              g.reshape(128))
    return _combine(aggp, denp, h, b0.reshape(N))


kernel = jax.jit(_latte)


# speedup vs baseline: 1.1443x; 1.1443x over previous
import functools

import jax
import jax.numpy as jnp
from jax import lax
from jax.experimental import pallas as pl
from jax.experimental.pallas import tpu as pltpu
from jax.experimental.pallas import tpu_sc as plsc

N = 10000
E = 320000
D = 128
NC = 2
NS = 16
L = 16
NW = NC * NS
E_W = E // NW
KB = 80
NBLK = E_W // KB
AGG_CH = 1000


def _prologue_body(x_ref, w_ref, a_ref, c_ref, p_ref,
                   h_ref, sl_ref, sr_ref, b0_ref, g_ref):
    x = x_ref[...]
    h = jnp.maximum(jnp.dot(x, w_ref[...], preferred_element_type=jnp.float32), 0.0)
    h_ref[...] = h
    s = jnp.dot(h, a_ref[...], preferred_element_type=jnp.float32)
    sl = s[:, 0:1] + p_ref[0]
    sr = s[:, 1:2] + p_ref[1]
    sl_ref[...] = sl
    sr_ref[...] = sr
    b = jnp.dot(x, c_ref[...], preferred_element_type=jnp.float32)
    b0_ref[...] = jax.nn.sigmoid(b[:, 0:1] + p_ref[2])
    m = jnp.max(sl) + jnp.max(sr)
    g = jnp.maximum(m, 0.2 * m)
    g_ref[...] = jnp.full((1, 128), g, jnp.float32)


def _prologue(x, w_lin, a_mat, c_mat, p):
    return pl.pallas_call(
        _prologue_body,
        out_shape=(
            jax.ShapeDtypeStruct((N, D), jnp.float32),
            jax.ShapeDtypeStruct((N, 1), jnp.float32),
            jax.ShapeDtypeStruct((N, 1), jnp.float32),
            jax.ShapeDtypeStruct((N, 1), jnp.float32),
            jax.ShapeDtypeStruct((1, 128), jnp.float32),
        ),
        in_specs=[
            pl.BlockSpec(memory_space=pltpu.VMEM),
            pl.BlockSpec(memory_space=pltpu.VMEM),
            pl.BlockSpec(memory_space=pltpu.VMEM),
            pl.BlockSpec(memory_space=pltpu.VMEM),
            pl.BlockSpec(memory_space=pltpu.SMEM),
        ],
    )(x, w_lin, a_mat, c_mat, p)


def _edge_body(h_hbm, sl_hbm, sr_hbm, ii_hbm, jj_hbm, g_hbm,
               aggp_hbm, denp_hbm,
               sh_agg, sh_den, sl_v, sr_v,
               ii0, jj0, e0, rows0, ii1, jj1, e1, rows1, g_v,
               isem0, gsem0, rsem0, dsem0, isem1, gsem1, rsem1, dsem1):
    cid = lax.axis_index("c")
    sid = lax.axis_index("s")
    wid = sid * NC + cid
    bufs = ((ii0, jj0, e0, rows0, isem0, gsem0, rsem0, dsem0),
            (ii1, jj1, e1, rows1, isem1, gsem1, rsem1, dsem1))

    pltpu.sync_copy(sl_hbm, sl_v)
    pltpu.sync_copy(sr_hbm, sr_v)
    pltpu.sync_copy(g_hbm.at[pl.ds(0, L)], g_v)

    def _issue_idx(b, s):
        ii_v, jj_v = bufs[s][0], bufs[s][1]
        isem = bufs[s][4]
        off = wid * E_W + b * KB
        pltpu.async_copy(ii_hbm.at[pl.ds(off, KB)], ii_v, isem)
        pltpu.async_copy(jj_hbm.at[pl.ds(off, KB)], jj_v, isem)

    def _zrow(r, _):
        for c in range(D // L):
            rows0[r, pl.ds(c * L, L)] = jnp.zeros((L,), jnp.float32)
        return 0
    lax.fori_loop(0, KB, _zrow, 0)
    for q in range(KB // L):
        e0[pl.ds(q * L, L)] = jnp.zeros((L,), jnp.float32)

    _issue_idx(0, 0)

    nch = N // KB
    for k in range((nch + NS - 1) // NS):
        c = sid + k * NS

        @pl.when(c < nch)
        def _():
            pltpu.sync_copy(rows0, sh_agg.at[pl.ds(c * KB, KB)])
            pltpu.sync_copy(e0, sh_den.at[pl.ds(c * KB, KB)])

    plsc.subcore_barrier()

    g16 = g_v[...]

    def _process(b, s, first):
        ii_v, jj_v, e_v, rows_v, isem, gsem, rsem, dsem = bufs[s]
        o = 1 - s
        ii_o, jj_o, e_o, rows_o, isem_o, gsem_o, rsem_o, dsem_o = bufs[o]
        off = wid * E_W + b * KB
        pltpu.make_async_copy(ii_hbm.at[pl.ds(off, KB)], ii_v, isem).wait()
        pltpu.make_async_copy(jj_hbm.at[pl.ds(off, KB)], jj_v, isem).wait()
        gcp = pltpu.async_copy(h_hbm.at[jj_v], rows_v, gsem)

        @plsc.parallel_loop(0, KB // L, unroll=2)
        def _escore(q):
            ii16 = ii_v[pl.ds(q * L, L)]
            jj16 = jj_v[pl.ds(q * L, L)]
            sv = (plsc.load_gather(sl_v, [ii16]) +
                  plsc.load_gather(sr_v, [jj16]))
            a = jnp.maximum(sv, 0.2 * sv)
            e_v[pl.ds(q * L, L)] = jnp.exp(a - g16)

        gcp.wait()

        @plsc.parallel_loop(0, KB // L, unroll=2)
        def _rows16(q):
            e16 = e_v[pl.ds(q * L, L)]
            for r in range(L):
                eb = jnp.take_along_axis(
                    e16, jnp.full((L,), r, jnp.int32), axis=0,
                    mode="promise_in_bounds")
                row = q * L + r
                for c in range(D // L):
                    rows_v[row, pl.ds(c * L, L)] = (
                        rows_v[row, pl.ds(c * L, L)] * eb)

        pltpu.async_copy(rows_v, sh_agg.at[ii_v], rsem, add=True)
        pltpu.async_copy(e_v, sh_den.at[ii_v], dsem, add=True)

        if not first:
            pltpu.make_async_copy(rows_o, sh_agg.at[ii_o], rsem_o).wait()
            pltpu.make_async_copy(e_o, sh_den.at[ii_o], dsem_o).wait()

        @pl.when(b + 1 < NBLK)
        def _():
            _issue_idx(b + 1, o)

    _process(0, 0, True)

    def _pair(p, _):
        _process(1 + 2 * p, 1, False)
        _process(2 + 2 * p, 0, False)
        return 0
    lax.fori_loop(0, (NBLK - 1) // 2, _pair, 0)

    pltpu.make_async_copy(rows0, sh_agg.at[ii0], rsem0).wait()
    pltpu.make_async_copy(e0, sh_den.at[ii0], dsem0).wait()

    plsc.subcore_barrier()

    @pl.when(sid < N // AGG_CH)
    def _():
        pltpu.sync_copy(sh_agg.at[pl.ds(sid * AGG_CH, AGG_CH)],
                        aggp_hbm.at[cid, pl.ds(sid * AGG_CH, AGG_CH)])

    @pl.when(sid == 0)
    def _():
        pltpu.sync_copy(sh_den, denp_hbm.at[cid])


def _edge_kernel(h, sl, sr, ii, jj, g):
    mesh = plsc.VectorSubcoreMesh(core_axis_name="c", subcore_axis_name="s",
                                  num_cores=NC, num_subcores=NS)
    return pl.kernel(
        _edge_body,
        out_type=(
            jax.ShapeDtypeStruct((NC, N, D), jnp.float32),
            jax.ShapeDtypeStruct((NC, N), jnp.float32),
        ),
        mesh=mesh,
        compiler_params=pltpu.CompilerParams(needs_layout_passes=False),
        scratch_types=(
            [
                pltpu.VMEM_SHARED((N, D), jnp.float32),
                pltpu.VMEM_SHARED((N,), jnp.float32),
                pltpu.VMEM((N,), jnp.float32),
                pltpu.VMEM((N,), jnp.float32),
            ]
            + 2 * [
                pltpu.VMEM((KB,), jnp.int32),
                pltpu.VMEM((KB,), jnp.int32),
                pltpu.VMEM((KB,), jnp.float32),
                pltpu.VMEM((KB, D), jnp.float32),
            ]
            + [pltpu.VMEM((L,), jnp.float32)]
            + 8 * [pltpu.SemaphoreType.DMA]
        ),
    )(h, sl, sr, ii, jj, g)


def _combine_body(aggp_ref, denp_ref, h_ref, b0_ref, out_ref):
    agg = aggp_ref[0] + aggp_ref[1]
    den = denp_ref[0] + denp_ref[1] + 1e-16
    b0 = b0_ref[...][:, None]
    out_ref[...] = b0 * (agg / den[:, None]) + (1.0 - b0) * h_ref[...]


def _combine(aggp, denp, h, b0):
    return pl.pallas_call(
        _combine_body,
        out_shape=jax.ShapeDtypeStruct((N, D), jnp.float32),
    )(aggp, denp, h, b0)


def _latte(x, x_index, edge_index, W_lin, conv_w, conv_b,
           attn_l_w, attn_l_b, attn_r_w, attn_r_b):
    p = jnp.stack([
        jnp.asarray(attn_l_b, jnp.float32),
        jnp.asarray(attn_r_b, jnp.float32),
        (conv_b[0] - conv_b[1]).astype(jnp.float32),
    ])
    a_mat = jnp.zeros((D, 128), jnp.float32)
    a_mat = a_mat.at[:, 0].set(attn_l_w).at[:, 1].set(attn_r_w)
    c_mat = jnp.zeros((D, 128), jnp.float32)
    c_mat = c_mat.at[:, 0].set(conv_w[0] - conv_w[1])
    h, sl, sr, b0, g = _prologue(x, W_lin, a_mat, c_mat, p)
    ii = edge_index[0]
    jj = edge_index[1]
    aggp, denp = _edge_kernel(h, sl.reshape(N), sr.reshape(N), ii, jj,
                              g.reshape(128))
    return _combine(aggp, denp, h, b0.reshape(N))


kernel = jax.jit(_latte)


# speedup vs baseline: 1.1470x; 1.0023x over previous
import functools

import jax
import jax.numpy as jnp
from jax import lax
from jax.experimental import pallas as pl
from jax.experimental.pallas import tpu as pltpu
from jax.experimental.pallas import tpu_sc as plsc

N = 10000
E = 320000
D = 128
NC = 2
NS = 16
L = 16
NW = NC * NS
E_W = E // NW
KB = 80
NBLK = E_W // KB
AGG_CH = 1000


def _prologue_body(x_ref, w_ref, a_ref, c_ref, p_ref,
                   h_ref, sl_ref, sr_ref, b0_ref, g_ref):
    x = x_ref[...]
    h = jnp.maximum(jnp.dot(x, w_ref[...], preferred_element_type=jnp.float32), 0.0)
    h_ref[...] = h
    s = jnp.dot(h, a_ref[...], preferred_element_type=jnp.float32)
    sl = s[:, 0:1] + p_ref[0]
    sr = s[:, 1:2] + p_ref[1]
    sl_ref[...] = sl
    sr_ref[...] = sr
    b = jnp.dot(x, c_ref[...], preferred_element_type=jnp.float32)
    b0_ref[...] = jax.nn.sigmoid(b[:, 0:1] + p_ref[2])
    m = jnp.max(sl) + jnp.max(sr)
    g = jnp.maximum(m, 0.2 * m)
    g_ref[...] = jnp.full((1, 128), g, jnp.float32)


def _prologue(x, w_lin, a_mat, c_mat, p):
    return pl.pallas_call(
        _prologue_body,
        out_shape=(
            jax.ShapeDtypeStruct((N, D), jnp.float32),
            jax.ShapeDtypeStruct((N, 1), jnp.float32),
            jax.ShapeDtypeStruct((N, 1), jnp.float32),
            jax.ShapeDtypeStruct((N, 1), jnp.float32),
            jax.ShapeDtypeStruct((1, 128), jnp.float32),
        ),
        in_specs=[
            pl.BlockSpec(memory_space=pltpu.VMEM),
            pl.BlockSpec(memory_space=pltpu.VMEM),
            pl.BlockSpec(memory_space=pltpu.VMEM),
            pl.BlockSpec(memory_space=pltpu.VMEM),
            pl.BlockSpec(memory_space=pltpu.SMEM),
        ],
    )(x, w_lin, a_mat, c_mat, p)


def _edge_body(h_hbm, sl_hbm, sr_hbm, ii_hbm, jj_hbm, g_hbm,
               aggp_hbm, denp_hbm,
               sh_agg, sh_den, sl_v, sr_v,
               ii0, jj0, e0, rows0, ii1, jj1, e1, rows1, g_v,
               isem0, gsem0, rsem0, dsem0, isem1, gsem1, rsem1, dsem1):
    cid = lax.axis_index("c")
    sid = lax.axis_index("s")
    wid = sid * NC + cid
    bufs = ((ii0, jj0, e0, rows0, isem0, gsem0, rsem0, dsem0),
            (ii1, jj1, e1, rows1, isem1, gsem1, rsem1, dsem1))

    pltpu.sync_copy(sl_hbm, sl_v)
    pltpu.sync_copy(sr_hbm, sr_v)
    pltpu.sync_copy(g_hbm.at[pl.ds(0, L)], g_v)

    def _issue_idx(b, s):
        ii_v, jj_v = bufs[s][0], bufs[s][1]
        isem = bufs[s][4]
        off = wid * E_W + b * KB
        pltpu.async_copy(ii_hbm.at[pl.ds(off, KB)], ii_v, isem)
        pltpu.async_copy(jj_hbm.at[pl.ds(off, KB)], jj_v, isem)

    def _zrow(r, _):
        for c in range(D // L):
            rows0[r, pl.ds(c * L, L)] = jnp.zeros((L,), jnp.float32)
        return 0
    lax.fori_loop(0, KB, _zrow, 0)
    for q in range(KB // L):
        e0[pl.ds(q * L, L)] = jnp.zeros((L,), jnp.float32)

    _issue_idx(0, 0)

    nch = N // KB
    for k in range((nch + NS - 1) // NS):
        c = sid + k * NS

        @pl.when(c < nch)
        def _():
            pltpu.sync_copy(rows0, sh_agg.at[pl.ds(c * KB, KB)])
            pltpu.sync_copy(e0, sh_den.at[pl.ds(c * KB, KB)])

    plsc.subcore_barrier()

    g16 = g_v[...]

    def _process(b, s, first):
        ii_v, jj_v, e_v, rows_v, isem, gsem, rsem, dsem = bufs[s]
        o = 1 - s
        ii_o, jj_o, e_o, rows_o, isem_o, gsem_o, rsem_o, dsem_o = bufs[o]
        off = wid * E_W + b * KB
        pltpu.make_async_copy(ii_hbm.at[pl.ds(off, KB)], ii_v, isem).wait()
        pltpu.make_async_copy(jj_hbm.at[pl.ds(off, KB)], jj_v, isem).wait()
        gcp = pltpu.async_copy(h_hbm.at[jj_v], rows_v, gsem)

        @plsc.parallel_loop(0, KB // L, unroll=2)
        def _escore(q):
            ii16 = ii_v[pl.ds(q * L, L)]
            jj16 = jj_v[pl.ds(q * L, L)]
            sv = (plsc.load_gather(sl_v, [ii16]) +
                  plsc.load_gather(sr_v, [jj16]))
            a = jnp.maximum(sv, 0.2 * sv)
            e_v[pl.ds(q * L, L)] = jnp.exp(a - g16)

        pltpu.async_copy(e_v, sh_den.at[ii_v], dsem, add=True)

        gcp.wait()

        @plsc.parallel_loop(0, KB // L, unroll=2)
        def _rows16(q):
            e16 = e_v[pl.ds(q * L, L)]
            for r in range(L):
                eb = jnp.take_along_axis(
                    e16, jnp.full((L,), r, jnp.int32), axis=0,
                    mode="promise_in_bounds")
                row = q * L + r
                for c in range(D // L):
                    rows_v[row, pl.ds(c * L, L)] = (
                        rows_v[row, pl.ds(c * L, L)] * eb)

        pltpu.async_copy(rows_v, sh_agg.at[ii_v], rsem, add=True)

        if not first:
            pltpu.make_async_copy(rows_o, sh_agg.at[ii_o], rsem_o).wait()
            pltpu.make_async_copy(e_o, sh_den.at[ii_o], dsem_o).wait()

        @pl.when(b + 1 < NBLK)
        def _():
            _issue_idx(b + 1, o)

    _process(0, 0, True)

    def _pair(p, _):
        _process(1 + 2 * p, 1, False)
        _process(2 + 2 * p, 0, False)
        return 0
    lax.fori_loop(0, (NBLK - 1) // 2, _pair, 0)

    pltpu.make_async_copy(rows0, sh_agg.at[ii0], rsem0).wait()
    pltpu.make_async_copy(e0, sh_den.at[ii0], dsem0).wait()

    plsc.subcore_barrier()

    @pl.when(sid < N // AGG_CH)
    def _():
        pltpu.sync_copy(sh_agg.at[pl.ds(sid * AGG_CH, AGG_CH)],
                        aggp_hbm.at[cid, pl.ds(sid * AGG_CH, AGG_CH)])

    @pl.when(sid == 0)
    def _():
        pltpu.sync_copy(sh_den, denp_hbm.at[cid])


def _edge_kernel(h, sl, sr, ii, jj, g):
    mesh = plsc.VectorSubcoreMesh(core_axis_name="c", subcore_axis_name="s",
                                  num_cores=NC, num_subcores=NS)
    return pl.kernel(
        _edge_body,
        out_type=(
            jax.ShapeDtypeStruct((NC, N, D), jnp.float32),
            jax.ShapeDtypeStruct((NC, N), jnp.float32),
        ),
        mesh=mesh,
        compiler_params=pltpu.CompilerParams(needs_layout_passes=False),
        scratch_types=(
            [
                pltpu.VMEM_SHARED((N, D), jnp.float32),
                pltpu.VMEM_SHARED((N,), jnp.float32),
                pltpu.VMEM((N,), jnp.float32),
                pltpu.VMEM((N,), jnp.float32),
            ]
            + 2 * [
                pltpu.VMEM((KB,), jnp.int32),
                pltpu.VMEM((KB,), jnp.int32),
                pltpu.VMEM((KB,), jnp.float32),
                pltpu.VMEM((KB, D), jnp.float32),
            ]
            + [pltpu.VMEM((L,), jnp.float32)]
            + 8 * [pltpu.SemaphoreType.DMA]
        ),
    )(h, sl, sr, ii, jj, g)


def _combine_body(aggp_ref, denp_ref, h_ref, b0_ref, out_ref):
    agg = aggp_ref[0] + aggp_ref[1]
    den = denp_ref[0] + denp_ref[1] + 1e-16
    b0 = b0_ref[...][:, None]
    out_ref[...] = b0 * (agg / den[:, None]) + (1.0 - b0) * h_ref[...]


def _combine(aggp, denp, h, b0):
    return pl.pallas_call(
        _combine_body,
        out_shape=jax.ShapeDtypeStruct((N, D), jnp.float32),
    )(aggp, denp, h, b0)


def _latte(x, x_index, edge_index, W_lin, conv_w, conv_b,
           attn_l_w, attn_l_b, attn_r_w, attn_r_b):
    p = jnp.stack([
        jnp.asarray(attn_l_b, jnp.float32),
        jnp.asarray(attn_r_b, jnp.float32),
        (conv_b[0] - conv_b[1]).astype(jnp.float32),
    ])
    a_mat = jnp.zeros((D, 128), jnp.float32)
    a_mat = a_mat.at[:, 0].set(attn_l_w).at[:, 1].set(attn_r_w)
    c_mat = jnp.zeros((D, 128), jnp.float32)
    c_mat = c_mat.at[:, 0].set(conv_w[0] - conv_w[1])
    h, sl, sr, b0, g = _prologue(x, W_lin, a_mat, c_mat, p)
    ii = edge_index[0]
    jj = edge_index[1]
    aggp, denp = _edge_kernel(h, sl.reshape(N), sr.reshape(N), ii, jj,
                              g.reshape(128))
    return _combine(aggp, denp, h, b0.reshape(N))


kernel = jax.jit(_latte)
